# baseline (speedup 1.0000x reference)
import functools
import jax
import jax.numpy as jnp
from jax import lax
from jax.experimental import pallas as pl
from jax.experimental.pallas import tpu as pltpu
from jax.experimental.pallas import tpu_sc as plsc

N = 10000
E = 160000
F_IN = 256
H1 = 8
C1 = 8
C2 = 16

NPAD = 10240
NW = 32
K = 128
CH = 40
EPT = CH * K
EPAD = NW * EPT
NSUB = 16
RPS = NPAD // NSUB

BR = 400
DEC_BR = 1024
DEC_BC = 4096


def _dense1_body(x_ref, w_ref, gs_ref, gd_ref, tsrc_ref, tdst_ref):
    h = jnp.dot(x_ref[...], w_ref[...], preferred_element_type=jnp.float32)
    asrc = jnp.dot(h, gs_ref[...], preferred_element_type=jnp.float32)
    adst = jnp.dot(h, gd_ref[...], preferred_element_type=jnp.float32)
    z8 = jnp.zeros((h.shape[0], 8), jnp.float32)
    tsrc_ref[...] = jnp.concatenate([h, asrc, z8], axis=1)
    tdst_ref[...] = jnp.concatenate([z8, adst], axis=1)


def _dense1(x, W1, Gs1, Gd1):
    grid = (N // BR,)
    return pl.pallas_call(
        _dense1_body,
        grid=grid,
        in_specs=[
            pl.BlockSpec((BR, F_IN), lambda i: (i, 0)),
            pl.BlockSpec((F_IN, 64), lambda i: (0, 0)),
            pl.BlockSpec((64, 8), lambda i: (0, 0)),
            pl.BlockSpec((64, 8), lambda i: (0, 0)),
        ],
        out_specs=[
            pl.BlockSpec((BR, 80), lambda i: (i, 0)),
            pl.BlockSpec((BR, 16), lambda i: (i, 0)),
        ],
        out_shape=[
            jax.ShapeDtypeStruct((N, 80), jnp.float32),
            jax.ShapeDtypeStruct((N, 16), jnp.float32),
        ],
    )(x, W1, Gs1, Gd1)


def _dense2_body(pa_ref, pb_ref, gout_ref, gexp_ref, b1_ref, w2_ref,
                 gs2_ref, gd2_ref, tsrc_ref, tdst_ref):
    S = pa_ref[...] + pb_ref[...]
    num = jnp.dot(S, gout_ref[...], preferred_element_type=jnp.float32)
    den = jnp.dot(S, gexp_ref[...], preferred_element_type=jnp.float32)
    t = num / (den + 1e-16) + b1_ref[...]
    h = jnp.where(t > 0, t, jnp.exp(jnp.minimum(t, 0.0)) - 1.0)
    h2 = jnp.dot(h, w2_ref[...], preferred_element_type=jnp.float32)
    a2s = jnp.dot(h2, gs2_ref[...], preferred_element_type=jnp.float32)
    a2d = jnp.dot(h2, gd2_ref[...], preferred_element_type=jnp.float32)
    z8 = jnp.zeros((S.shape[0], 8), jnp.float32)
    tsrc_ref[...] = jnp.concatenate([h2, a2s, z8], axis=1)
    tdst_ref[...] = jnp.concatenate([a2d, z8], axis=1)


def _dense2(Pa, Pb, Gout1, Gexp1, b1r, W2, Gs2, Gd2):
    grid = (N // BR,)
    return pl.pallas_call(
        _dense2_body,
        grid=grid,
        in_specs=[
            pl.BlockSpec((BR, 72), lambda i: (i, 0)),
            pl.BlockSpec((BR, 72), lambda i: (i, 0)),
            pl.BlockSpec((72, 64), lambda i: (0, 0)),
            pl.BlockSpec((72, 64), lambda i: (0, 0)),
            pl.BlockSpec((1, 64), lambda i: (0, 0)),
            pl.BlockSpec((64, 16), lambda i: (0, 0)),
            pl.BlockSpec((16, 8), lambda i: (0, 0)),
            pl.BlockSpec((16, 8), lambda i: (0, 0)),
        ],
        out_specs=[
            pl.BlockSpec((BR, 32), lambda i: (i, 0)),
            pl.BlockSpec((BR, 16), lambda i: (i, 0)),
        ],
        out_shape=[
            jax.ShapeDtypeStruct((N, 32), jnp.float32),
            jax.ShapeDtypeStruct((N, 16), jnp.float32),
        ],
    )(Pa, Pb, Gout1, Gexp1, b1r, W2, Gs2, Gd2)


def _dense3_body(pa_ref, pb_ref, g1_ref, g2_ref, b2_ref, z_ref):
    S = pa_ref[...] + pb_ref[...]
    num = jnp.dot(S, g1_ref[...], preferred_element_type=jnp.float32)
    den = jnp.dot(S, g2_ref[...], preferred_element_type=jnp.float32)
    z_ref[...] = num / (den + 1e-16) + b2_ref[...]


def _dense3(Pa, Pb, G1, G2, b2r):
    grid = (N // BR,)
    return pl.pallas_call(
        _dense3_body,
        grid=grid,
        in_specs=[
            pl.BlockSpec((BR, 32), lambda i: (i, 0)),
            pl.BlockSpec((BR, 32), lambda i: (i, 0)),
            pl.BlockSpec((32, 16), lambda i: (0, 0)),
            pl.BlockSpec((32, 16), lambda i: (0, 0)),
            pl.BlockSpec((1, 16), lambda i: (0, 0)),
        ],
        out_specs=pl.BlockSpec((BR, 16), lambda i: (i, 0)),
        out_shape=jax.ShapeDtypeStruct((N, 16), jnp.float32),
    )(Pa, Pb, G1, G2, b2r)


def _decoder_body(z_row_ref, z_col_ref, logp_ref, prod_ref):
    j = pl.program_id(1)
    zr = z_row_ref[...]
    zc = z_col_ref[...]
    prod_ref[...] = jax.lax.dot_general(
        zr, zc, (((1,), (1,)), ((), ())), preferred_element_type=jnp.float32)

    @pl.when(j == 0)
    def _():
        m = jnp.max(zr, axis=1, keepdims=True)
        s = jnp.sum(jnp.exp(zr - m), axis=1, keepdims=True)
        logp_ref[...] = zr - m - jnp.log(s)


def _decoder(z):
    grid = (pl.cdiv(N, DEC_BR), pl.cdiv(N, DEC_BC))
    return pl.pallas_call(
        _decoder_body,
        grid=grid,
        in_specs=[
            pl.BlockSpec((DEC_BR, C2), lambda i, j: (i, 0)),
            pl.BlockSpec((DEC_BC, C2), lambda i, j: (j, 0)),
        ],
        out_specs=[
            pl.BlockSpec((DEC_BR, C2), lambda i, j: (i, 0)),
            pl.BlockSpec((DEC_BR, DEC_BC), lambda i, j: (i, j)),
        ],
        out_shape=[
            jax.ShapeDtypeStruct((N, C2), jnp.float32),
            jax.ShapeDtypeStruct((N, N), jnp.float32),
        ],
    )(z, z)


def _make_edge_kernel(H, HC, RS, RM):
    SA = min(HC, RM - 16)
    mesh = plsc.VectorSubcoreMesh(core_axis_name="c", subcore_axis_name="s",
                                  num_cores=2, num_subcores=16)

    @functools.partial(
        pl.kernel,
        out_type=jax.ShapeDtypeStruct((2, NPAD, RM), jnp.float32),
        mesh=mesh,
        scratch_types=[
            pltpu.VMEM((CH, K), jnp.int32),
            pltpu.VMEM((CH, K), jnp.int32),
            pltpu.VMEM((K, RS), jnp.float32),
            pltpu.VMEM((K, RS), jnp.float32),
            pltpu.VMEM((K, 16), jnp.float32),
            pltpu.VMEM((K, 16), jnp.float32),
            pltpu.VMEM((K, RM), jnp.float32),
            pltpu.VMEM((64, RM), jnp.float32),
            pltpu.VMEM_SHARED((NPAD, RM), jnp.float32),
            pltpu.SemaphoreType.DMA,
            pltpu.SemaphoreType.DMA,
            pltpu.SemaphoreType.DMA,
            pltpu.SemaphoreType.DMA,
        ],
        compiler_params=pltpu.CompilerParams(
            needs_layout_passes=False, use_tc_tiling_on_sc=False),
    )
    def edge_kernel(src_hbm, dst_hbm, tsrc_hbm, tdst_hbm, out_hbm,
                    sidx, didx, arow0, arow1, brow0, brow1, mrow, stage, accum,
                    sem1, sem2, sem3, sem4):
        c = lax.axis_index("c")
        sub = lax.axis_index("s")
        wid = sub * 2 + c
        zv = jnp.zeros((16,), jnp.float32)
        zoffs = sorted({k * 16 for k in range(RM // 16)} | {RM - 16})

        def zb(r, _):
            for o in zoffs:
                stage[r, pl.ds(o, 16)] = zv
            return 0
        lax.fori_loop(0, 64, zb, 0)

        def zcopy(t, _):
            pltpu.sync_copy(stage, accum.at[pl.ds(sub * RPS + t * 64, 64)])
            return 0
        lax.fori_loop(0, RPS // 64, zcopy, 0)
        plsc.subcore_barrier()

        pltpu.sync_copy(src_hbm.at[wid], sidx)
        pltpu.sync_copy(dst_hbm.at[wid], didx)
        iota = lax.iota(jnp.int32, 16)

        def issue(ci, ar, br, s1, s2):
            pltpu.async_copy(tsrc_hbm.at[sidx.at[ci]], ar, s1)
            pltpu.async_copy(tdst_hbm.at[didx.at[ci]], br, s2)

        def wait(ci, ar, br, s1, s2):
            pltpu.make_async_copy(tsrc_hbm.at[sidx.at[ci]], ar, s1).wait()
            pltpu.make_async_copy(tdst_hbm.at[didx.at[ci]], br, s2).wait()

        def compute(ci, ar, br):
            @plsc.parallel_loop(0, K, step=1, unroll=4)
            def ph(e):
                a16 = ar[e, pl.ds(SA, 16)]
                b16 = br[e, pl.ds(0, 16)]
                xv = a16 + b16
                sv = jnp.exp(jnp.maximum(xv, 0.2 * xv))
                mrow[e, pl.ds(SA, 16)] = sv
                er = jnp.full((16,), e, jnp.int32)
                for j in range(HC // 16):
                    if H == 1:
                        scol = jnp.full((16,), HC, jnp.int32)
                    else:
                        scol = HC + 2 * j + (iota >= 8).astype(jnp.int32)
                    svb = plsc.load_gather(mrow, [er, scol])
                    hv = ar[e, pl.ds(16 * j, 16)]
                    mrow[e, pl.ds(16 * j, 16)] = hv * svb

            pltpu.sync_copy(mrow, accum.at[didx.at[ci]], add=True)

        issue(0, arow0, brow0, sem1, sem2)

        def chunk2(cj, _):
            cia = 2 * cj
            cib = 2 * cj + 1
            nxt = lax.rem(cia + 2, CH)
            wait(cia, arow0, brow0, sem1, sem2)
            issue(cib, arow1, brow1, sem3, sem4)
            compute(cia, arow0, brow0)
            wait(cib, arow1, brow1, sem3, sem4)
            issue(nxt, arow0, brow0, sem1, sem2)
            compute(cib, arow1, brow1)
            return 0
        lax.fori_loop(0, CH // 2, chunk2, 0)
        wait(0, arow0, brow0, sem1, sem2)

        plsc.subcore_barrier()
        pltpu.sync_copy(accum.at[pl.ds(sub * RPS, RPS)],
                        out_hbm.at[c, pl.ds(sub * RPS, RPS)])

    return edge_kernel


def _pad_nodes(t):
    return jnp.pad(t, ((0, NPAD - N), (0, 0)))


def kernel(x, edge_index, W1, a_src1, a_dst1, b1, W2, a_src2, a_dst2, b2):
    eye8r = jnp.repeat(jnp.eye(8, dtype=jnp.float32), 8, axis=0)
    Gs1 = eye8r * a_src1.reshape(64, 1)
    Gd1 = eye8r * a_dst1.reshape(64, 1)
    Gout1 = jnp.eye(72, 64, dtype=jnp.float32)
    Gexp1 = jnp.concatenate([
        jnp.zeros((64, 64), jnp.float32),
        jnp.repeat(jnp.eye(8, dtype=jnp.float32), 8, axis=1),
    ], axis=0)
    Gs2 = jnp.concatenate([a_src2.T, jnp.zeros((16, 7), jnp.float32)], axis=1)
    Gd2 = jnp.concatenate([a_dst2.T, jnp.zeros((16, 7), jnp.float32)], axis=1)
    G1 = jnp.eye(32, 16, dtype=jnp.float32)
    G2 = jnp.zeros((32, 16), jnp.float32).at[16, :].set(1.0)
    b1r = b1.reshape(1, 64)
    b2r = b2.reshape(1, 16)

    src = edge_index[0].astype(jnp.int32)
    dst = edge_index[1].astype(jnp.int32)
    trash = NPAD - 1
    src_r = jnp.pad(src, (0, EPAD - E), constant_values=trash).reshape(NW, CH, K)
    dst_r = jnp.pad(dst, (0, EPAD - E), constant_values=trash).reshape(NW, CH, K)

    _edges1 = _make_edge_kernel(H1, H1 * C1, 80, 72)
    _edges2 = _make_edge_kernel(1, C2, 32, 32)

    Tsrc1, Tdst1 = _dense1(x, W1, Gs1, Gd1)
    P1 = _edges1(src_r, dst_r, _pad_nodes(Tsrc1), _pad_nodes(Tdst1))
    Tsrc2, Tdst2 = _dense2(P1[0, :N], P1[1, :N], Gout1, Gexp1, b1r, W2, Gs2, Gd2)

    P2 = _edges2(src_r, dst_r, _pad_nodes(Tsrc2), _pad_nodes(Tdst2))
    z = _dense3(P2[0, :N], P2[1, :N], G1, G2, b2r)

    logp, prod = _decoder(z)
    return (logp, prod)


# speedup vs baseline: 1.0257x; 1.0257x over previous
import functools
import jax
import jax.numpy as jnp
from jax import lax
from jax.experimental import pallas as pl
from jax.experimental.pallas import tpu as pltpu
from jax.experimental.pallas import tpu_sc as plsc

N = 10000
E = 160000
F_IN = 256
H1 = 8
C1 = 8
C2 = 16

NPAD = 10240
NW = 32
K = 128
CH = 40
EPT = CH * K
EPAD = NW * EPT
NSUB = 16
RPS = NPAD // NSUB

BR = 400
DEC_BR = 2048
DEC_BC = 2560


def _dense1_body(x_ref, w_ref, gs_ref, gd_ref, tsrc_ref, tdst_ref):
    h = jnp.dot(x_ref[...], w_ref[...], preferred_element_type=jnp.float32)
    asrc = jnp.dot(h, gs_ref[...], preferred_element_type=jnp.float32)
    adst = jnp.dot(h, gd_ref[...], preferred_element_type=jnp.float32)
    z8 = jnp.zeros((h.shape[0], 8), jnp.float32)
    tsrc_ref[...] = jnp.concatenate([h, asrc, z8], axis=1)
    tdst_ref[...] = jnp.concatenate([z8, adst], axis=1)


def _dense1(x, W1, Gs1, Gd1):
    grid = (N // BR,)
    return pl.pallas_call(
        _dense1_body,
        grid=grid,
        in_specs=[
            pl.BlockSpec((BR, F_IN), lambda i: (i, 0)),
            pl.BlockSpec((F_IN, 64), lambda i: (0, 0)),
            pl.BlockSpec((64, 8), lambda i: (0, 0)),
            pl.BlockSpec((64, 8), lambda i: (0, 0)),
        ],
        out_specs=[
            pl.BlockSpec((BR, 80), lambda i: (i, 0)),
            pl.BlockSpec((BR, 16), lambda i: (i, 0)),
        ],
        out_shape=[
            jax.ShapeDtypeStruct((N, 80), jnp.float32),
            jax.ShapeDtypeStruct((N, 16), jnp.float32),
        ],
    )(x, W1, Gs1, Gd1)


def _dense2_body(pa_ref, pb_ref, gout_ref, gexp_ref, b1_ref, w2_ref,
                 gs2_ref, gd2_ref, tsrc_ref, tdst_ref):
    S = pa_ref[...] + pb_ref[...]
    num = jnp.dot(S, gout_ref[...], preferred_element_type=jnp.float32)
    den = jnp.dot(S, gexp_ref[...], preferred_element_type=jnp.float32)
    t = num / (den + 1e-16) + b1_ref[...]
    h = jnp.where(t > 0, t, jnp.exp(jnp.minimum(t, 0.0)) - 1.0)
    h2 = jnp.dot(h, w2_ref[...], preferred_element_type=jnp.float32)
    a2s = jnp.dot(h2, gs2_ref[...], preferred_element_type=jnp.float32)
    a2d = jnp.dot(h2, gd2_ref[...], preferred_element_type=jnp.float32)
    z8 = jnp.zeros((S.shape[0], 8), jnp.float32)
    tsrc_ref[...] = jnp.concatenate([h2, a2s, z8], axis=1)
    tdst_ref[...] = jnp.concatenate([a2d, z8], axis=1)


def _dense2(Pa, Pb, Gout1, Gexp1, b1r, W2, Gs2, Gd2):
    grid = (N // BR,)
    return pl.pallas_call(
        _dense2_body,
        grid=grid,
        in_specs=[
            pl.BlockSpec((BR, 72), lambda i: (i, 0)),
            pl.BlockSpec((BR, 72), lambda i: (i, 0)),
            pl.BlockSpec((72, 64), lambda i: (0, 0)),
            pl.BlockSpec((72, 64), lambda i: (0, 0)),
            pl.BlockSpec((1, 64), lambda i: (0, 0)),
            pl.BlockSpec((64, 16), lambda i: (0, 0)),
            pl.BlockSpec((16, 8), lambda i: (0, 0)),
            pl.BlockSpec((16, 8), lambda i: (0, 0)),
        ],
        out_specs=[
            pl.BlockSpec((BR, 32), lambda i: (i, 0)),
            pl.BlockSpec((BR, 16), lambda i: (i, 0)),
        ],
        out_shape=[
            jax.ShapeDtypeStruct((N, 32), jnp.float32),
            jax.ShapeDtypeStruct((N, 16), jnp.float32),
        ],
    )(Pa, Pb, Gout1, Gexp1, b1r, W2, Gs2, Gd2)


def _dense3_body(pa_ref, pb_ref, g1_ref, g2_ref, b2_ref, z_ref):
    S = pa_ref[...] + pb_ref[...]
    num = jnp.dot(S, g1_ref[...], preferred_element_type=jnp.float32)
    den = jnp.dot(S, g2_ref[...], preferred_element_type=jnp.float32)
    z_ref[...] = num / (den + 1e-16) + b2_ref[...]


def _dense3(Pa, Pb, G1, G2, b2r):
    grid = (N // BR,)
    return pl.pallas_call(
        _dense3_body,
        grid=grid,
        in_specs=[
            pl.BlockSpec((BR, 32), lambda i: (i, 0)),
            pl.BlockSpec((BR, 32), lambda i: (i, 0)),
            pl.BlockSpec((32, 16), lambda i: (0, 0)),
            pl.BlockSpec((32, 16), lambda i: (0, 0)),
            pl.BlockSpec((1, 16), lambda i: (0, 0)),
        ],
        out_specs=pl.BlockSpec((BR, 16), lambda i: (i, 0)),
        out_shape=jax.ShapeDtypeStruct((N, 16), jnp.float32),
    )(Pa, Pb, G1, G2, b2r)


def _decoder_body(z_row_ref, z_col_ref, logp_ref, prod_ref):
    j = pl.program_id(1)
    zr = z_row_ref[...]
    zc = z_col_ref[...]
    prod_ref[...] = jax.lax.dot_general(
        zr, zc, (((1,), (1,)), ((), ())), preferred_element_type=jnp.float32)

    @pl.when(j == 0)
    def _():
        m = jnp.max(zr, axis=1, keepdims=True)
        s = jnp.sum(jnp.exp(zr - m), axis=1, keepdims=True)
        logp_ref[...] = zr - m - jnp.log(s)


def _decoder(z):
    grid = (pl.cdiv(N, DEC_BR), pl.cdiv(N, DEC_BC))
    return pl.pallas_call(
        _decoder_body,
        grid=grid,
        in_specs=[
            pl.BlockSpec((DEC_BR, C2), lambda i, j: (i, 0)),
            pl.BlockSpec((DEC_BC, C2), lambda i, j: (j, 0)),
        ],
        out_specs=[
            pl.BlockSpec((DEC_BR, C2), lambda i, j: (i, 0)),
            pl.BlockSpec((DEC_BR, DEC_BC), lambda i, j: (i, j)),
        ],
        out_shape=[
            jax.ShapeDtypeStruct((N, C2), jnp.float32),
            jax.ShapeDtypeStruct((N, N), jnp.float32),
        ],
    )(z, z)


def _make_edge_kernel(H, HC, RS, RM):
    SA = min(HC, RM - 16)
    mesh = plsc.VectorSubcoreMesh(core_axis_name="c", subcore_axis_name="s",
                                  num_cores=2, num_subcores=16)

    @functools.partial(
        pl.kernel,
        out_type=jax.ShapeDtypeStruct((2, NPAD, RM), jnp.float32),
        mesh=mesh,
        scratch_types=[
            pltpu.VMEM((CH, K), jnp.int32),
            pltpu.VMEM((CH, K), jnp.int32),
            pltpu.VMEM((K, RS), jnp.float32),
            pltpu.VMEM((K, RS), jnp.float32),
            pltpu.VMEM((K, 16), jnp.float32),
            pltpu.VMEM((K, 16), jnp.float32),
            pltpu.VMEM((K, RM), jnp.float32),
            pltpu.VMEM((64, RM), jnp.float32),
            pltpu.VMEM_SHARED((NPAD, RM), jnp.float32),
            pltpu.SemaphoreType.DMA,
            pltpu.SemaphoreType.DMA,
            pltpu.SemaphoreType.DMA,
            pltpu.SemaphoreType.DMA,
        ],
        compiler_params=pltpu.CompilerParams(
            needs_layout_passes=False, use_tc_tiling_on_sc=False),
    )
    def edge_kernel(src_hbm, dst_hbm, tsrc_hbm, tdst_hbm, out_hbm,
                    sidx, didx, arow0, arow1, brow0, brow1, mrow, stage, accum,
                    sem1, sem2, sem3, sem4):
        c = lax.axis_index("c")
        sub = lax.axis_index("s")
        wid = sub * 2 + c
        zv = jnp.zeros((16,), jnp.float32)
        zoffs = sorted({k * 16 for k in range(RM // 16)} | {RM - 16})

        def zb(r, _):
            for o in zoffs:
                stage[r, pl.ds(o, 16)] = zv
            return 0
        lax.fori_loop(0, 64, zb, 0)

        def zcopy(t, _):
            pltpu.sync_copy(stage, accum.at[pl.ds(sub * RPS + t * 64, 64)])
            return 0
        lax.fori_loop(0, RPS // 64, zcopy, 0)
        plsc.subcore_barrier()

        pltpu.sync_copy(src_hbm.at[wid], sidx)
        pltpu.sync_copy(dst_hbm.at[wid], didx)
        iota = lax.iota(jnp.int32, 16)

        def issue(ci, ar, br, s1, s2):
            pltpu.async_copy(tsrc_hbm.at[sidx.at[ci]], ar, s1)
            pltpu.async_copy(tdst_hbm.at[didx.at[ci]], br, s2)

        def wait(ci, ar, br, s1, s2):
            pltpu.make_async_copy(tsrc_hbm.at[sidx.at[ci]], ar, s1).wait()
            pltpu.make_async_copy(tdst_hbm.at[didx.at[ci]], br, s2).wait()

        def compute(ci, ar, br):
            @plsc.parallel_loop(0, K, step=1, unroll=4)
            def ph(e):
                a16 = ar[e, pl.ds(SA, 16)]
                b16 = br[e, pl.ds(0, 16)]
                xv = a16 + b16
                sv = jnp.exp(jnp.maximum(xv, 0.2 * xv))
                mrow[e, pl.ds(SA, 16)] = sv
                er = jnp.full((16,), e, jnp.int32)
                for j in range(HC // 16):
                    if H == 1:
                        scol = jnp.full((16,), HC, jnp.int32)
                    else:
                        scol = HC + 2 * j + (iota >= 8).astype(jnp.int32)
                    svb = plsc.load_gather(mrow, [er, scol])
                    hv = ar[e, pl.ds(16 * j, 16)]
                    mrow[e, pl.ds(16 * j, 16)] = hv * svb

            pltpu.sync_copy(mrow, accum.at[didx.at[ci]], add=True)

        issue(0, arow0, brow0, sem1, sem2)

        def chunk2(cj, _):
            cia = 2 * cj
            cib = 2 * cj + 1
            nxt = lax.rem(cia + 2, CH)
            wait(cia, arow0, brow0, sem1, sem2)
            issue(cib, arow1, brow1, sem3, sem4)
            compute(cia, arow0, brow0)
            wait(cib, arow1, brow1, sem3, sem4)
            issue(nxt, arow0, brow0, sem1, sem2)
            compute(cib, arow1, brow1)
            return 0
        lax.fori_loop(0, CH // 2, chunk2, 0)
        wait(0, arow0, brow0, sem1, sem2)

        plsc.subcore_barrier()
        pltpu.sync_copy(accum.at[pl.ds(sub * RPS, RPS)],
                        out_hbm.at[c, pl.ds(sub * RPS, RPS)])

    return edge_kernel


def _pad_nodes(t):
    return jnp.pad(t, ((0, NPAD - N), (0, 0)))


def kernel(x, edge_index, W1, a_src1, a_dst1, b1, W2, a_src2, a_dst2, b2):
    eye8r = jnp.repeat(jnp.eye(8, dtype=jnp.float32), 8, axis=0)
    Gs1 = eye8r * a_src1.reshape(64, 1)
    Gd1 = eye8r * a_dst1.reshape(64, 1)
    Gout1 = jnp.eye(72, 64, dtype=jnp.float32)
    Gexp1 = jnp.concatenate([
        jnp.zeros((64, 64), jnp.float32),
        jnp.repeat(jnp.eye(8, dtype=jnp.float32), 8, axis=1),
    ], axis=0)
    Gs2 = jnp.concatenate([a_src2.T, jnp.zeros((16, 7), jnp.float32)], axis=1)
    Gd2 = jnp.concatenate([a_dst2.T, jnp.zeros((16, 7), jnp.float32)], axis=1)
    G1 = jnp.eye(32, 16, dtype=jnp.float32)
    G2 = jnp.zeros((32, 16), jnp.float32).at[16, :].set(1.0)
    b1r = b1.reshape(1, 64)
    b2r = b2.reshape(1, 16)

    src = edge_index[0].astype(jnp.int32)
    dst = edge_index[1].astype(jnp.int32)
    trash = NPAD - 1
    src_r = jnp.pad(src, (0, EPAD - E), constant_values=trash).reshape(NW, CH, K)
    dst_r = jnp.pad(dst, (0, EPAD - E), constant_values=trash).reshape(NW, CH, K)

    _edges1 = _make_edge_kernel(H1, H1 * C1, 80, 72)
    _edges2 = _make_edge_kernel(1, C2, 32, 32)

    Tsrc1, Tdst1 = _dense1(x, W1, Gs1, Gd1)
    P1 = _edges1(src_r, dst_r, _pad_nodes(Tsrc1), _pad_nodes(Tdst1))
    Tsrc2, Tdst2 = _dense2(P1[0, :N], P1[1, :N], Gout1, Gexp1, b1r, W2, Gs2, Gd2)

    P2 = _edges2(src_r, dst_r, _pad_nodes(Tsrc2), _pad_nodes(Tdst2))
    z = _dense3(P2[0, :N], P2[1, :N], G1, G2, b2r)

    logp, prod = _decoder(z)
    return (logp, prod)


# speedup vs baseline: 1.4084x; 1.3731x over previous
import functools
import jax
import jax.numpy as jnp
from jax import lax
from jax.experimental import pallas as pl
from jax.experimental.pallas import tpu as pltpu
from jax.experimental.pallas import tpu_sc as plsc

N = 10000
E = 160000
F_IN = 256
H1 = 8
C1 = 8
C2 = 16

NPAD = 10240
NW = 32
K = 128
CH = 40
EPT = CH * K
EPAD = NW * EPT
NSUB = 16
RPS = NPAD // NSUB

BR = 400
DEC_BR = 2048
DEC_BC = 2560


def _dense1_body(x_ref, w_ref, gs_ref, gd_ref, tsrc_ref, tdst_ref):
    h = jnp.dot(x_ref[...], w_ref[...], preferred_element_type=jnp.float32)
    asrc = jnp.dot(h, gs_ref[...], preferred_element_type=jnp.float32)
    adst = jnp.dot(h, gd_ref[...], preferred_element_type=jnp.float32)
    z8 = jnp.zeros((h.shape[0], 8), jnp.float32)
    tsrc_ref[...] = jnp.concatenate([h, asrc, z8], axis=1)
    tdst_ref[...] = jnp.concatenate([z8, adst], axis=1)


def _dense1(x, W1, Gs1, Gd1):
    grid = (N // BR,)
    return pl.pallas_call(
        _dense1_body,
        grid=grid,
        in_specs=[
            pl.BlockSpec((BR, F_IN), lambda i: (i, 0)),
            pl.BlockSpec((F_IN, 64), lambda i: (0, 0)),
            pl.BlockSpec((64, 8), lambda i: (0, 0)),
            pl.BlockSpec((64, 8), lambda i: (0, 0)),
        ],
        out_specs=[
            pl.BlockSpec((BR, 80), lambda i: (i, 0)),
            pl.BlockSpec((BR, 16), lambda i: (i, 0)),
        ],
        out_shape=[
            jax.ShapeDtypeStruct((N, 80), jnp.float32),
            jax.ShapeDtypeStruct((N, 16), jnp.float32),
        ],
    )(x, W1, Gs1, Gd1)


def _dense2_body(pa_ref, pb_ref, gout_ref, gexp_ref, b1_ref, w2_ref,
                 gs2_ref, gd2_ref, tsrc_ref, tdst_ref):
    S = pa_ref[...] + pb_ref[...]
    num = jnp.dot(S, gout_ref[...], preferred_element_type=jnp.float32)
    den = jnp.dot(S, gexp_ref[...], preferred_element_type=jnp.float32)
    t = num / (den + 1e-16) + b1_ref[...]
    h = jnp.where(t > 0, t, jnp.exp(jnp.minimum(t, 0.0)) - 1.0)
    h2 = jnp.dot(h, w2_ref[...], preferred_element_type=jnp.float32)
    a2s = jnp.dot(h2, gs2_ref[...], preferred_element_type=jnp.float32)
    a2d = jnp.dot(h2, gd2_ref[...], preferred_element_type=jnp.float32)
    z8 = jnp.zeros((S.shape[0], 8), jnp.float32)
    tsrc_ref[...] = jnp.concatenate([h2, a2s, z8], axis=1)
    tdst_ref[...] = jnp.concatenate([a2d, z8], axis=1)


def _dense2(Pa, Pb, Gout1, Gexp1, b1r, W2, Gs2, Gd2):
    grid = (N // BR,)
    return pl.pallas_call(
        _dense2_body,
        grid=grid,
        in_specs=[
            pl.BlockSpec((BR, 72), lambda i: (i, 0)),
            pl.BlockSpec((BR, 72), lambda i: (i, 0)),
            pl.BlockSpec((72, 64), lambda i: (0, 0)),
            pl.BlockSpec((72, 64), lambda i: (0, 0)),
            pl.BlockSpec((1, 64), lambda i: (0, 0)),
            pl.BlockSpec((64, 16), lambda i: (0, 0)),
            pl.BlockSpec((16, 8), lambda i: (0, 0)),
            pl.BlockSpec((16, 8), lambda i: (0, 0)),
        ],
        out_specs=[
            pl.BlockSpec((BR, 32), lambda i: (i, 0)),
            pl.BlockSpec((BR, 16), lambda i: (i, 0)),
        ],
        out_shape=[
            jax.ShapeDtypeStruct((N, 32), jnp.float32),
            jax.ShapeDtypeStruct((N, 16), jnp.float32),
        ],
    )(Pa, Pb, Gout1, Gexp1, b1r, W2, Gs2, Gd2)


def _dense3_body(pa_ref, pb_ref, g1_ref, g2_ref, b2_ref, z_ref):
    S = pa_ref[...] + pb_ref[...]
    num = jnp.dot(S, g1_ref[...], preferred_element_type=jnp.float32)
    den = jnp.dot(S, g2_ref[...], preferred_element_type=jnp.float32)
    z_ref[...] = num / (den + 1e-16) + b2_ref[...]


def _dense3(Pa, Pb, G1, G2, b2r):
    grid = (N // BR,)
    return pl.pallas_call(
        _dense3_body,
        grid=grid,
        in_specs=[
            pl.BlockSpec((BR, 32), lambda i: (i, 0)),
            pl.BlockSpec((BR, 32), lambda i: (i, 0)),
            pl.BlockSpec((32, 16), lambda i: (0, 0)),
            pl.BlockSpec((32, 16), lambda i: (0, 0)),
            pl.BlockSpec((1, 16), lambda i: (0, 0)),
        ],
        out_specs=pl.BlockSpec((BR, 16), lambda i: (i, 0)),
        out_shape=jax.ShapeDtypeStruct((N, 16), jnp.float32),
    )(Pa, Pb, G1, G2, b2r)


def _decoder_body(z_row_ref, z_col_ref, logp_ref, prod_ref):
    j = pl.program_id(1)
    zr = z_row_ref[...]
    zc = z_col_ref[...]
    prod_ref[...] = jax.lax.dot_general(
        zr, zc, (((1,), (1,)), ((), ())), preferred_element_type=jnp.float32)

    @pl.when(j == 0)
    def _():
        m = jnp.max(zr, axis=1, keepdims=True)
        s = jnp.sum(jnp.exp(zr - m), axis=1, keepdims=True)
        logp_ref[...] = zr - m - jnp.log(s)


def _decoder(z):
    grid = (pl.cdiv(N, DEC_BR), pl.cdiv(N, DEC_BC))
    return pl.pallas_call(
        _decoder_body,
        grid=grid,
        in_specs=[
            pl.BlockSpec((DEC_BR, C2), lambda i, j: (i, 0)),
            pl.BlockSpec((DEC_BC, C2), lambda i, j: (j, 0)),
        ],
        out_specs=[
            pl.BlockSpec((DEC_BR, C2), lambda i, j: (i, 0)),
            pl.BlockSpec((DEC_BR, DEC_BC), lambda i, j: (i, j)),
        ],
        out_shape=[
            jax.ShapeDtypeStruct((N, C2), jnp.float32),
            jax.ShapeDtypeStruct((N, N), jnp.float32),
        ],
    )(z, z)


def _make_edge_kernel(H, HC, RS, RM):
    SA = min(HC, RM - 16)
    mesh = plsc.VectorSubcoreMesh(core_axis_name="c", subcore_axis_name="s",
                                  num_cores=2, num_subcores=16)

    @functools.partial(
        pl.kernel,
        out_type=jax.ShapeDtypeStruct((2, NPAD, RM), jnp.float32),
        mesh=mesh,
        scratch_types=[
            pltpu.VMEM((CH, K), jnp.int32),
            pltpu.VMEM((CH, K), jnp.int32),
            pltpu.VMEM((K, RS), jnp.float32),
            pltpu.VMEM((K, RS), jnp.float32),
            pltpu.VMEM((K, 16), jnp.float32),
            pltpu.VMEM((K, 16), jnp.float32),
            pltpu.VMEM((K, RM), jnp.float32),
            pltpu.VMEM((64, RM), jnp.float32),
            pltpu.VMEM_SHARED((NPAD, RM), jnp.float32),
            pltpu.SemaphoreType.DMA,
            pltpu.SemaphoreType.DMA,
            pltpu.SemaphoreType.DMA,
            pltpu.SemaphoreType.DMA,
        ],
        compiler_params=pltpu.CompilerParams(
            needs_layout_passes=False, use_tc_tiling_on_sc=False),
    )
    def edge_kernel(src_hbm, dst_hbm, tsrc_hbm, tdst_hbm, out_hbm,
                    sidx, didx, arow0, arow1, brow0, brow1, mrow, stage, accum,
                    sem1, sem2, sem3, sem4):
        c = lax.axis_index("c")
        sub = lax.axis_index("s")
        wid = sub * 2 + c
        zv = jnp.zeros((16,), jnp.float32)
        zoffs = sorted({k * 16 for k in range(RM // 16)} | {RM - 16})

        def zb(r, _):
            for o in zoffs:
                stage[r, pl.ds(o, 16)] = zv
            return 0
        lax.fori_loop(0, 64, zb, 0)

        def zcopy(t, _):
            pltpu.sync_copy(stage, accum.at[pl.ds(sub * RPS + t * 64, 64)])
            return 0
        lax.fori_loop(0, RPS // 64, zcopy, 0)
        plsc.subcore_barrier()

        pltpu.sync_copy(src_hbm.at[wid], sidx)
        pltpu.sync_copy(dst_hbm.at[wid], didx)
        iota = lax.iota(jnp.int32, 16)

        def issue(ci, ar, br, s1, s2):
            pltpu.async_copy(tsrc_hbm.at[sidx.at[ci]], ar, s1)
            pltpu.async_copy(tdst_hbm.at[didx.at[ci]], br, s2)

        def wait(ci, ar, br, s1, s2):
            pltpu.make_async_copy(tsrc_hbm.at[sidx.at[ci]], ar, s1).wait()
            pltpu.make_async_copy(tdst_hbm.at[didx.at[ci]], br, s2).wait()

        def compute(ci, ar, br):
            @plsc.parallel_loop(0, K, step=1, unroll=4)
            def ph(e):
                a16 = ar[e, pl.ds(SA, 16)]
                b16 = br[e, pl.ds(0, 16)]
                xv = a16 + b16
                sv = jnp.exp(jnp.maximum(xv, 0.2 * xv))
                mrow[e, pl.ds(SA, 16)] = sv
                er = jnp.full((16,), e, jnp.int32)
                for j in range(HC // 16):
                    if H == 1:
                        scol = jnp.full((16,), HC, jnp.int32)
                    else:
                        scol = HC + 2 * j + (iota >= 8).astype(jnp.int32)
                    svb = plsc.load_gather(mrow, [er, scol])
                    hv = ar[e, pl.ds(16 * j, 16)]
                    mrow[e, pl.ds(16 * j, 16)] = hv * svb

            pltpu.sync_copy(mrow, accum.at[didx.at[ci]], add=True)

        issue(0, arow0, brow0, sem1, sem2)

        def chunk2(cj, _):
            cia = 2 * cj
            cib = 2 * cj + 1
            nxt = lax.rem(cia + 2, CH)
            wait(cia, arow0, brow0, sem1, sem2)
            issue(cib, arow1, brow1, sem3, sem4)
            compute(cia, arow0, brow0)
            wait(cib, arow1, brow1, sem3, sem4)
            issue(nxt, arow0, brow0, sem1, sem2)
            compute(cib, arow1, brow1)
            return 0
        lax.fori_loop(0, CH // 2, chunk2, 0)
        wait(0, arow0, brow0, sem1, sem2)

        plsc.subcore_barrier()
        pltpu.sync_copy(accum.at[pl.ds(sub * RPS, RPS)],
                        out_hbm.at[c, pl.ds(sub * RPS, RPS)])

    return edge_kernel


def _pad_nodes(t):
    return jnp.pad(t, ((0, NPAD - N), (0, 0)))


def kernel(x, edge_index, W1, a_src1, a_dst1, b1, W2, a_src2, a_dst2, b2):
    eye8r = jnp.repeat(jnp.eye(8, dtype=jnp.float32), 8, axis=0)
    Gs1 = eye8r * a_src1.reshape(64, 1)
    Gd1 = eye8r * a_dst1.reshape(64, 1)
    Gout1 = jnp.eye(72, 64, dtype=jnp.float32)
    Gexp1 = jnp.concatenate([
        jnp.zeros((64, 64), jnp.float32),
        jnp.repeat(jnp.eye(8, dtype=jnp.float32), 8, axis=1),
    ], axis=0)
    Gs2 = jnp.concatenate([a_src2.T, jnp.zeros((16, 7), jnp.float32)], axis=1)
    Gd2 = jnp.concatenate([a_dst2.T, jnp.zeros((16, 7), jnp.float32)], axis=1)
    G1 = jnp.eye(32, 16, dtype=jnp.float32)
    G2 = jnp.zeros((32, 16), jnp.float32).at[16, :].set(1.0)
    b1r = b1.reshape(1, 64)
    b2r = b2.reshape(1, 16)

    src = edge_index[0].astype(jnp.int32)
    dst = edge_index[1].astype(jnp.int32)
    epw = E // NW
    ppw = EPT - epw
    trash = (N + jnp.arange(NW * ppw, dtype=jnp.int32) % (NPAD - N)
             ).reshape(NW, ppw)
    src_r = jnp.concatenate([src.reshape(NW, epw), trash], axis=1
                            ).reshape(NW, CH, K)
    dst_r = jnp.concatenate([dst.reshape(NW, epw), trash], axis=1
                            ).reshape(NW, CH, K)

    _edges1 = _make_edge_kernel(H1, H1 * C1, 80, 72)
    _edges2 = _make_edge_kernel(1, C2, 32, 32)

    Tsrc1, Tdst1 = _dense1(x, W1, Gs1, Gd1)
    P1 = _edges1(src_r, dst_r, _pad_nodes(Tsrc1), _pad_nodes(Tdst1))
    Tsrc2, Tdst2 = _dense2(P1[0, :N], P1[1, :N], Gout1, Gexp1, b1r, W2, Gs2, Gd2)

    P2 = _edges2(src_r, dst_r, _pad_nodes(Tsrc2), _pad_nodes(Tdst2))
    z = _dense3(P2[0, :N], P2[1, :N], G1, G2, b2r)

    logp, prod = _decoder(z)
    return (logp, prod)


# speedup vs baseline: 1.5463x; 1.0979x over previous
import functools
import jax
import jax.numpy as jnp
from jax import lax
from jax.experimental import pallas as pl
from jax.experimental.pallas import tpu as pltpu
from jax.experimental.pallas import tpu_sc as plsc

N = 10000
E = 160000
F_IN = 256
H1 = 8
C1 = 8
C2 = 16

NPAD = 10240
NW = 32
K = 128
CH = 40
EPT = CH * K
EPAD = NW * EPT
NSUB = 16
RPS = NPAD // NSUB

BR = 2000
DEC_BR = 2048
DEC_BC = 2560


def _dense1_body(x_ref, w_ref, gs_ref, gd_ref, tsrc_ref, tdst_ref):
    h = jnp.dot(x_ref[...], w_ref[...], preferred_element_type=jnp.float32)
    asrc = jnp.dot(h, gs_ref[...], preferred_element_type=jnp.float32)
    adst = jnp.dot(h, gd_ref[...], preferred_element_type=jnp.float32)
    z8 = jnp.zeros((h.shape[0], 8), jnp.float32)
    tsrc_ref[...] = jnp.concatenate([h, asrc, z8], axis=1)
    tdst_ref[...] = jnp.concatenate([z8, adst], axis=1)


def _dense1(x, W1, Gs1, Gd1):
    grid = (N // BR,)
    return pl.pallas_call(
        _dense1_body,
        grid=grid,
        in_specs=[
            pl.BlockSpec((BR, F_IN), lambda i: (i, 0)),
            pl.BlockSpec((F_IN, 64), lambda i: (0, 0)),
            pl.BlockSpec((64, 8), lambda i: (0, 0)),
            pl.BlockSpec((64, 8), lambda i: (0, 0)),
        ],
        out_specs=[
            pl.BlockSpec((BR, 80), lambda i: (i, 0)),
            pl.BlockSpec((BR, 16), lambda i: (i, 0)),
        ],
        out_shape=[
            jax.ShapeDtypeStruct((N, 80), jnp.float32),
            jax.ShapeDtypeStruct((N, 16), jnp.float32),
        ],
    )(x, W1, Gs1, Gd1)


def _dense2_body(pa_ref, pb_ref, gout_ref, gexp_ref, b1_ref, w2_ref,
                 gs2_ref, gd2_ref, tsrc_ref, tdst_ref):
    S = pa_ref[...] + pb_ref[...]
    num = jnp.dot(S, gout_ref[...], preferred_element_type=jnp.float32)
    den = jnp.dot(S, gexp_ref[...], preferred_element_type=jnp.float32)
    t = num / (den + 1e-16) + b1_ref[...]
    h = jnp.where(t > 0, t, jnp.exp(jnp.minimum(t, 0.0)) - 1.0)
    h2 = jnp.dot(h, w2_ref[...], preferred_element_type=jnp.float32)
    a2s = jnp.dot(h2, gs2_ref[...], preferred_element_type=jnp.float32)
    a2d = jnp.dot(h2, gd2_ref[...], preferred_element_type=jnp.float32)
    z8 = jnp.zeros((S.shape[0], 8), jnp.float32)
    tsrc_ref[...] = jnp.concatenate([h2, a2s, z8], axis=1)
    tdst_ref[...] = jnp.concatenate([a2d, z8], axis=1)


def _dense2(Pa, Pb, Gout1, Gexp1, b1r, W2, Gs2, Gd2):
    grid = (N // BR,)
    return pl.pallas_call(
        _dense2_body,
        grid=grid,
        in_specs=[
            pl.BlockSpec((BR, 72), lambda i: (i, 0)),
            pl.BlockSpec((BR, 72), lambda i: (i, 0)),
            pl.BlockSpec((72, 64), lambda i: (0, 0)),
            pl.BlockSpec((72, 64), lambda i: (0, 0)),
            pl.BlockSpec((1, 64), lambda i: (0, 0)),
            pl.BlockSpec((64, 16), lambda i: (0, 0)),
            pl.BlockSpec((16, 8), lambda i: (0, 0)),
            pl.BlockSpec((16, 8), lambda i: (0, 0)),
        ],
        out_specs=[
            pl.BlockSpec((BR, 32), lambda i: (i, 0)),
            pl.BlockSpec((BR, 16), lambda i: (i, 0)),
        ],
        out_shape=[
            jax.ShapeDtypeStruct((N, 32), jnp.float32),
            jax.ShapeDtypeStruct((N, 16), jnp.float32),
        ],
    )(Pa, Pb, Gout1, Gexp1, b1r, W2, Gs2, Gd2)


def _dense3_body(pa_ref, pb_ref, g1_ref, g2_ref, b2_ref, z_ref):
    S = pa_ref[...] + pb_ref[...]
    num = jnp.dot(S, g1_ref[...], preferred_element_type=jnp.float32)
    den = jnp.dot(S, g2_ref[...], preferred_element_type=jnp.float32)
    z_ref[...] = num / (den + 1e-16) + b2_ref[...]


def _dense3(Pa, Pb, G1, G2, b2r):
    grid = (N // BR,)
    return pl.pallas_call(
        _dense3_body,
        grid=grid,
        in_specs=[
            pl.BlockSpec((BR, 32), lambda i: (i, 0)),
            pl.BlockSpec((BR, 32), lambda i: (i, 0)),
            pl.BlockSpec((32, 16), lambda i: (0, 0)),
            pl.BlockSpec((32, 16), lambda i: (0, 0)),
            pl.BlockSpec((1, 16), lambda i: (0, 0)),
        ],
        out_specs=pl.BlockSpec((BR, 16), lambda i: (i, 0)),
        out_shape=jax.ShapeDtypeStruct((N, 16), jnp.float32),
    )(Pa, Pb, G1, G2, b2r)


def _decoder_body(z_row_ref, z_col_ref, logp_ref, prod_ref):
    j = pl.program_id(1)
    zr = z_row_ref[...]
    zc = z_col_ref[...]
    prod_ref[...] = jax.lax.dot_general(
        zr, zc, (((1,), (1,)), ((), ())), preferred_element_type=jnp.float32)

    @pl.when(j == 0)
    def _():
        m = jnp.max(zr, axis=1, keepdims=True)
        s = jnp.sum(jnp.exp(zr - m), axis=1, keepdims=True)
        logp_ref[...] = zr - m - jnp.log(s)


def _decoder(z):
    grid = (pl.cdiv(N, DEC_BR), pl.cdiv(N, DEC_BC))
    return pl.pallas_call(
        _decoder_body,
        grid=grid,
        in_specs=[
            pl.BlockSpec((DEC_BR, C2), lambda i, j: (i, 0)),
            pl.BlockSpec((DEC_BC, C2), lambda i, j: (j, 0)),
        ],
        out_specs=[
            pl.BlockSpec((DEC_BR, C2), lambda i, j: (i, 0)),
            pl.BlockSpec((DEC_BR, DEC_BC), lambda i, j: (i, j)),
        ],
        out_shape=[
            jax.ShapeDtypeStruct((N, C2), jnp.float32),
            jax.ShapeDtypeStruct((N, N), jnp.float32),
        ],
    )(z, z)


def _make_edge_kernel(H, HC, RS, RM):
    SA = min(HC, RM - 16)
    mesh = plsc.VectorSubcoreMesh(core_axis_name="c", subcore_axis_name="s",
                                  num_cores=2, num_subcores=16)

    @functools.partial(
        pl.kernel,
        out_type=jax.ShapeDtypeStruct((2, NPAD, RM), jnp.float32),
        mesh=mesh,
        scratch_types=[
            pltpu.VMEM((CH, K), jnp.int32),
            pltpu.VMEM((CH, K), jnp.int32),
            pltpu.VMEM((K, RS), jnp.float32),
            pltpu.VMEM((K, RS), jnp.float32),
            pltpu.VMEM((K, 16), jnp.float32),
            pltpu.VMEM((K, 16), jnp.float32),
            pltpu.VMEM((K, RM), jnp.float32),
            pltpu.VMEM((64, RM), jnp.float32),
            pltpu.VMEM_SHARED((NPAD, RM), jnp.float32),
            pltpu.SemaphoreType.DMA,
            pltpu.SemaphoreType.DMA,
            pltpu.SemaphoreType.DMA,
            pltpu.SemaphoreType.DMA,
        ],
        compiler_params=pltpu.CompilerParams(
            needs_layout_passes=False, use_tc_tiling_on_sc=False),
    )
    def edge_kernel(src_hbm, dst_hbm, tsrc_hbm, tdst_hbm, out_hbm,
                    sidx, didx, arow0, arow1, brow0, brow1, mrow, stage, accum,
                    sem1, sem2, sem3, sem4):
        c = lax.axis_index("c")
        sub = lax.axis_index("s")
        wid = sub * 2 + c
        zv = jnp.zeros((16,), jnp.float32)
        zoffs = sorted({k * 16 for k in range(RM // 16)} | {RM - 16})

        def zb(r, _):
            for o in zoffs:
                stage[r, pl.ds(o, 16)] = zv
            return 0
        lax.fori_loop(0, 64, zb, 0)

        def zcopy(t, _):
            pltpu.sync_copy(stage, accum.at[pl.ds(sub * RPS + t * 64, 64)])
            return 0
        lax.fori_loop(0, RPS // 64, zcopy, 0)
        plsc.subcore_barrier()

        pltpu.sync_copy(src_hbm.at[wid], sidx)
        pltpu.sync_copy(dst_hbm.at[wid], didx)
        iota = lax.iota(jnp.int32, 16)

        def issue(ci, ar, br, s1, s2):
            pltpu.async_copy(tsrc_hbm.at[sidx.at[ci]], ar, s1)
            pltpu.async_copy(tdst_hbm.at[didx.at[ci]], br, s2)

        def wait(ci, ar, br, s1, s2):
            pltpu.make_async_copy(tsrc_hbm.at[sidx.at[ci]], ar, s1).wait()
            pltpu.make_async_copy(tdst_hbm.at[didx.at[ci]], br, s2).wait()

        def compute(ci, ar, br):
            @plsc.parallel_loop(0, K, step=1, unroll=4)
            def ph(e):
                a16 = ar[e, pl.ds(SA, 16)]
                b16 = br[e, pl.ds(0, 16)]
                xv = a16 + b16
                sv = jnp.exp(jnp.maximum(xv, 0.2 * xv))
                mrow[e, pl.ds(SA, 16)] = sv
                er = jnp.full((16,), e, jnp.int32)
                for j in range(HC // 16):
                    if H == 1:
                        scol = jnp.full((16,), HC, jnp.int32)
                    else:
                        scol = HC + 2 * j + (iota >= 8).astype(jnp.int32)
                    svb = plsc.load_gather(mrow, [er, scol])
                    hv = ar[e, pl.ds(16 * j, 16)]
                    mrow[e, pl.ds(16 * j, 16)] = hv * svb

            pltpu.sync_copy(mrow, accum.at[didx.at[ci]], add=True)

        issue(0, arow0, brow0, sem1, sem2)

        def chunk2(cj, _):
            cia = 2 * cj
            cib = 2 * cj + 1
            nxt = lax.rem(cia + 2, CH)
            wait(cia, arow0, brow0, sem1, sem2)
            issue(cib, arow1, brow1, sem3, sem4)
            compute(cia, arow0, brow0)
            wait(cib, arow1, brow1, sem3, sem4)
            issue(nxt, arow0, brow0, sem1, sem2)
            compute(cib, arow1, brow1)
            return 0
        lax.fori_loop(0, CH // 2, chunk2, 0)
        wait(0, arow0, brow0, sem1, sem2)

        plsc.subcore_barrier()
        pltpu.sync_copy(accum.at[pl.ds(sub * RPS, RPS)],
                        out_hbm.at[c, pl.ds(sub * RPS, RPS)])

    return edge_kernel


def _pad_nodes(t):
    return jnp.pad(t, ((0, NPAD - N), (0, 0)))


def kernel(x, edge_index, W1, a_src1, a_dst1, b1, W2, a_src2, a_dst2, b2):
    eye8r = jnp.repeat(jnp.eye(8, dtype=jnp.float32), 8, axis=0)
    Gs1 = eye8r * a_src1.reshape(64, 1)
    Gd1 = eye8r * a_dst1.reshape(64, 1)
    Gout1 = jnp.eye(72, 64, dtype=jnp.float32)
    Gexp1 = jnp.concatenate([
        jnp.zeros((64, 64), jnp.float32),
        jnp.repeat(jnp.eye(8, dtype=jnp.float32), 8, axis=1),
    ], axis=0)
    Gs2 = jnp.concatenate([a_src2.T, jnp.zeros((16, 7), jnp.float32)], axis=1)
    Gd2 = jnp.concatenate([a_dst2.T, jnp.zeros((16, 7), jnp.float32)], axis=1)
    G1 = jnp.eye(32, 16, dtype=jnp.float32)
    G2 = jnp.zeros((32, 16), jnp.float32).at[16, :].set(1.0)
    b1r = b1.reshape(1, 64)
    b2r = b2.reshape(1, 16)

    src = edge_index[0].astype(jnp.int32)
    dst = edge_index[1].astype(jnp.int32)
    epw = E // NW
    ppw = EPT - epw
    trash = (N + jnp.arange(NW * ppw, dtype=jnp.int32) % (NPAD - N)
             ).reshape(NW, ppw)
    src_r = jnp.concatenate([src.reshape(NW, epw), trash], axis=1
                            ).reshape(NW, CH, K)
    dst_r = jnp.concatenate([dst.reshape(NW, epw), trash], axis=1
                            ).reshape(NW, CH, K)

    _edges1 = _make_edge_kernel(H1, H1 * C1, 80, 72)
    _edges2 = _make_edge_kernel(1, C2, 32, 32)

    Tsrc1, Tdst1 = _dense1(x, W1, Gs1, Gd1)
    P1 = _edges1(src_r, dst_r, _pad_nodes(Tsrc1), _pad_nodes(Tdst1))
    Tsrc2, Tdst2 = _dense2(P1[0, :N], P1[1, :N], Gout1, Gexp1, b1r, W2, Gs2, Gd2)

    P2 = _edges2(src_r, dst_r, _pad_nodes(Tsrc2), _pad_nodes(Tdst2))
    z = _dense3(P2[0, :N], P2[1, :N], G1, G2, b2r)

    logp, prod = _decoder(z)
    return (logp, prod)


# speedup vs baseline: 1.6091x; 1.0406x over previous
import functools
import jax
import jax.numpy as jnp
from jax import lax
from jax.experimental import pallas as pl
from jax.experimental.pallas import tpu as pltpu
from jax.experimental.pallas import tpu_sc as plsc

N = 10000
E = 160000
F_IN = 256
H1 = 8
C1 = 8
C2 = 16

NPAD = 10240
NW = 32
K = 128
CH = 40
EPT = CH * K
EPAD = NW * EPT
NSUB = 16
RPS = NPAD // NSUB

BR = 2000
DEC_BR = 2048
DEC_BC = 2560


def _dense1_body(x_ref, w_ref, gs_ref, gd_ref, tsrc_ref, tdst_ref):
    h = jnp.dot(x_ref[...], w_ref[...], preferred_element_type=jnp.float32)
    asrc = jnp.dot(h, gs_ref[...], preferred_element_type=jnp.float32)
    adst = jnp.dot(h, gd_ref[...], preferred_element_type=jnp.float32)
    z8 = jnp.zeros((h.shape[0], 8), jnp.float32)
    tsrc_ref[...] = jnp.concatenate([h, asrc, z8], axis=1)
    tdst_ref[...] = jnp.concatenate([z8, adst], axis=1)


def _dense1(x, W1, Gs1, Gd1):
    grid = (N // BR,)
    return pl.pallas_call(
        _dense1_body,
        grid=grid,
        in_specs=[
            pl.BlockSpec((BR, F_IN), lambda i: (i, 0)),
            pl.BlockSpec((F_IN, 64), lambda i: (0, 0)),
            pl.BlockSpec((64, 8), lambda i: (0, 0)),
            pl.BlockSpec((64, 8), lambda i: (0, 0)),
        ],
        out_specs=[
            pl.BlockSpec((BR, 80), lambda i: (i, 0)),
            pl.BlockSpec((BR, 16), lambda i: (i, 0)),
        ],
        out_shape=[
            jax.ShapeDtypeStruct((N, 80), jnp.float32),
            jax.ShapeDtypeStruct((N, 16), jnp.float32),
        ],
    )(x, W1, Gs1, Gd1)


def _dense2_body(pa_ref, pb_ref, gout_ref, gexp_ref, b1_ref, w2_ref,
                 gs2_ref, gd2_ref, tsrc_ref, tdst_ref):
    S = pa_ref[0] + pb_ref[0]
    num = jnp.dot(S, gout_ref[...], preferred_element_type=jnp.float32)
    den = jnp.dot(S, gexp_ref[...], preferred_element_type=jnp.float32)
    t = num / (den + 1e-16) + b1_ref[...]
    h = jnp.where(t > 0, t, jnp.exp(jnp.minimum(t, 0.0)) - 1.0)
    h2 = jnp.dot(h, w2_ref[...], preferred_element_type=jnp.float32)
    a2s = jnp.dot(h2, gs2_ref[...], preferred_element_type=jnp.float32)
    a2d = jnp.dot(h2, gd2_ref[...], preferred_element_type=jnp.float32)
    z8 = jnp.zeros((S.shape[0], 8), jnp.float32)
    tsrc_ref[...] = jnp.concatenate([h2, a2s, z8], axis=1)
    tdst_ref[...] = jnp.concatenate([a2d, z8], axis=1)


def _dense2(P, Gout1, Gexp1, b1r, W2, Gs2, Gd2):
    grid = (N // BR,)
    return pl.pallas_call(
        _dense2_body,
        grid=grid,
        in_specs=[
            pl.BlockSpec((1, BR, 72), lambda i: (0, i, 0)),
            pl.BlockSpec((1, BR, 72), lambda i: (1, i, 0)),
            pl.BlockSpec((72, 64), lambda i: (0, 0)),
            pl.BlockSpec((72, 64), lambda i: (0, 0)),
            pl.BlockSpec((1, 64), lambda i: (0, 0)),
            pl.BlockSpec((64, 16), lambda i: (0, 0)),
            pl.BlockSpec((16, 8), lambda i: (0, 0)),
            pl.BlockSpec((16, 8), lambda i: (0, 0)),
        ],
        out_specs=[
            pl.BlockSpec((BR, 32), lambda i: (i, 0)),
            pl.BlockSpec((BR, 16), lambda i: (i, 0)),
        ],
        out_shape=[
            jax.ShapeDtypeStruct((N, 32), jnp.float32),
            jax.ShapeDtypeStruct((N, 16), jnp.float32),
        ],
    )(P, P, Gout1, Gexp1, b1r, W2, Gs2, Gd2)


def _dense3_body(pa_ref, pb_ref, g1_ref, g2_ref, b2_ref, z_ref):
    S = pa_ref[0] + pb_ref[0]
    num = jnp.dot(S, g1_ref[...], preferred_element_type=jnp.float32)
    den = jnp.dot(S, g2_ref[...], preferred_element_type=jnp.float32)
    z_ref[...] = num / (den + 1e-16) + b2_ref[...]


def _dense3(P, G1, G2, b2r):
    grid = (N // BR,)
    return pl.pallas_call(
        _dense3_body,
        grid=grid,
        in_specs=[
            pl.BlockSpec((1, BR, 32), lambda i: (0, i, 0)),
            pl.BlockSpec((1, BR, 32), lambda i: (1, i, 0)),
            pl.BlockSpec((32, 16), lambda i: (0, 0)),
            pl.BlockSpec((32, 16), lambda i: (0, 0)),
            pl.BlockSpec((1, 16), lambda i: (0, 0)),
        ],
        out_specs=pl.BlockSpec((BR, 16), lambda i: (i, 0)),
        out_shape=jax.ShapeDtypeStruct((N, 16), jnp.float32),
    )(P, P, G1, G2, b2r)


def _decoder_body(z_row_ref, z_col_ref, logp_ref, prod_ref):
    j = pl.program_id(1)
    zr = z_row_ref[...]
    zc = z_col_ref[...]
    prod_ref[...] = jax.lax.dot_general(
        zr, zc, (((1,), (1,)), ((), ())), preferred_element_type=jnp.float32)

    @pl.when(j == 0)
    def _():
        m = jnp.max(zr, axis=1, keepdims=True)
        s = jnp.sum(jnp.exp(zr - m), axis=1, keepdims=True)
        logp_ref[...] = zr - m - jnp.log(s)


def _decoder(z):
    grid = (pl.cdiv(N, DEC_BR), pl.cdiv(N, DEC_BC))
    return pl.pallas_call(
        _decoder_body,
        grid=grid,
        in_specs=[
            pl.BlockSpec((DEC_BR, C2), lambda i, j: (i, 0)),
            pl.BlockSpec((DEC_BC, C2), lambda i, j: (j, 0)),
        ],
        out_specs=[
            pl.BlockSpec((DEC_BR, C2), lambda i, j: (i, 0)),
            pl.BlockSpec((DEC_BR, DEC_BC), lambda i, j: (i, j)),
        ],
        out_shape=[
            jax.ShapeDtypeStruct((N, C2), jnp.float32),
            jax.ShapeDtypeStruct((N, N), jnp.float32),
        ],
    )(z, z)


def _make_edge_kernel(H, HC, RS, RM):
    SA = min(HC, RM - 16)
    mesh = plsc.VectorSubcoreMesh(core_axis_name="c", subcore_axis_name="s",
                                  num_cores=2, num_subcores=16)

    @functools.partial(
        pl.kernel,
        out_type=jax.ShapeDtypeStruct((2, NPAD, RM), jnp.float32),
        mesh=mesh,
        scratch_types=[
            pltpu.VMEM((CH, K), jnp.int32),
            pltpu.VMEM((CH, K), jnp.int32),
            pltpu.VMEM((K, RS), jnp.float32),
            pltpu.VMEM((K, RS), jnp.float32),
            pltpu.VMEM((K, 16), jnp.float32),
            pltpu.VMEM((K, 16), jnp.float32),
            pltpu.VMEM((K, RM), jnp.float32),
            pltpu.VMEM((64, RM), jnp.float32),
            pltpu.VMEM_SHARED((NPAD, RM), jnp.float32),
            pltpu.SemaphoreType.DMA,
            pltpu.SemaphoreType.DMA,
            pltpu.SemaphoreType.DMA,
            pltpu.SemaphoreType.DMA,
        ],
        compiler_params=pltpu.CompilerParams(
            needs_layout_passes=False, use_tc_tiling_on_sc=False),
    )
    def edge_kernel(src_hbm, dst_hbm, tsrc_hbm, tdst_hbm, out_hbm,
                    sidx, didx, arow0, arow1, brow0, brow1, mrow, stage, accum,
                    sem1, sem2, sem3, sem4):
        c = lax.axis_index("c")
        sub = lax.axis_index("s")
        wid = sub * 2 + c
        zv = jnp.zeros((16,), jnp.float32)
        zoffs = sorted({k * 16 for k in range(RM // 16)} | {RM - 16})

        def zb(r, _):
            for o in zoffs:
                stage[r, pl.ds(o, 16)] = zv
            return 0
        lax.fori_loop(0, 64, zb, 0)

        def zcopy(t, _):
            pltpu.sync_copy(stage, accum.at[pl.ds(sub * RPS + t * 64, 64)])
            return 0
        lax.fori_loop(0, RPS // 64, zcopy, 0)
        plsc.subcore_barrier()

        pltpu.sync_copy(src_hbm.at[wid], sidx)
        pltpu.sync_copy(dst_hbm.at[wid], didx)
        iota = lax.iota(jnp.int32, 16)

        def issue(ci, ar, br, s1, s2):
            pltpu.async_copy(tsrc_hbm.at[sidx.at[ci]], ar, s1)
            pltpu.async_copy(tdst_hbm.at[didx.at[ci]], br, s2)

        def wait(ci, ar, br, s1, s2):
            pltpu.make_async_copy(tsrc_hbm.at[sidx.at[ci]], ar, s1).wait()
            pltpu.make_async_copy(tdst_hbm.at[didx.at[ci]], br, s2).wait()

        def compute(ci, ar, br):
            @plsc.parallel_loop(0, K, step=1, unroll=4)
            def ph(e):
                a16 = ar[e, pl.ds(SA, 16)]
                b16 = br[e, pl.ds(0, 16)]
                xv = a16 + b16
                sv = jnp.exp(jnp.maximum(xv, 0.2 * xv))
                mrow[e, pl.ds(SA, 16)] = sv
                er = jnp.full((16,), e, jnp.int32)
                for j in range(HC // 16):
                    if H == 1:
                        scol = jnp.full((16,), HC, jnp.int32)
                    else:
                        scol = HC + 2 * j + (iota >= 8).astype(jnp.int32)
                    svb = plsc.load_gather(mrow, [er, scol])
                    hv = ar[e, pl.ds(16 * j, 16)]
                    mrow[e, pl.ds(16 * j, 16)] = hv * svb

            pltpu.sync_copy(mrow, accum.at[didx.at[ci]], add=True)

        issue(0, arow0, brow0, sem1, sem2)

        def chunk2(cj, _):
            cia = 2 * cj
            cib = 2 * cj + 1
            nxt = lax.rem(cia + 2, CH)
            wait(cia, arow0, brow0, sem1, sem2)
            issue(cib, arow1, brow1, sem3, sem4)
            compute(cia, arow0, brow0)
            wait(cib, arow1, brow1, sem3, sem4)
            issue(nxt, arow0, brow0, sem1, sem2)
            compute(cib, arow1, brow1)
            return 0
        lax.fori_loop(0, CH // 2, chunk2, 0)
        wait(0, arow0, brow0, sem1, sem2)

        plsc.subcore_barrier()
        pltpu.sync_copy(accum.at[pl.ds(sub * RPS, RPS)],
                        out_hbm.at[c, pl.ds(sub * RPS, RPS)])

    return edge_kernel


def _pad_nodes(t):
    return jnp.pad(t, ((0, NPAD - N), (0, 0)))


def kernel(x, edge_index, W1, a_src1, a_dst1, b1, W2, a_src2, a_dst2, b2):
    eye8r = jnp.repeat(jnp.eye(8, dtype=jnp.float32), 8, axis=0)
    Gs1 = eye8r * a_src1.reshape(64, 1)
    Gd1 = eye8r * a_dst1.reshape(64, 1)
    Gout1 = jnp.eye(72, 64, dtype=jnp.float32)
    Gexp1 = jnp.concatenate([
        jnp.zeros((64, 64), jnp.float32),
        jnp.repeat(jnp.eye(8, dtype=jnp.float32), 8, axis=1),
    ], axis=0)
    Gs2 = jnp.concatenate([a_src2.T, jnp.zeros((16, 7), jnp.float32)], axis=1)
    Gd2 = jnp.concatenate([a_dst2.T, jnp.zeros((16, 7), jnp.float32)], axis=1)
    G1 = jnp.eye(32, 16, dtype=jnp.float32)
    G2 = jnp.zeros((32, 16), jnp.float32).at[16, :].set(1.0)
    b1r = b1.reshape(1, 64)
    b2r = b2.reshape(1, 16)

    src = edge_index[0].astype(jnp.int32)
    dst = edge_index[1].astype(jnp.int32)
    epw = E // NW
    ppw = EPT - epw
    trash = (N + jnp.arange(NW * ppw, dtype=jnp.int32) % (NPAD - N)
             ).reshape(NW, ppw)
    src_r = jnp.concatenate([src.reshape(NW, epw), trash], axis=1
                            ).reshape(NW, CH, K)
    dst_r = jnp.concatenate([dst.reshape(NW, epw), trash], axis=1
                            ).reshape(NW, CH, K)

    _edges1 = _make_edge_kernel(H1, H1 * C1, 80, 72)
    _edges2 = _make_edge_kernel(1, C2, 32, 32)

    Tsrc1, Tdst1 = _dense1(x, W1, Gs1, Gd1)
    P1 = _edges1(src_r, dst_r, _pad_nodes(Tsrc1), _pad_nodes(Tdst1))
    Tsrc2, Tdst2 = _dense2(P1, Gout1, Gexp1, b1r, W2, Gs2, Gd2)

    P2 = _edges2(src_r, dst_r, _pad_nodes(Tsrc2), _pad_nodes(Tdst2))
    z = _dense3(P2, G1, G2, b2r)

    logp, prod = _decoder(z)
    return (logp, prod)
